# Initial kernel scaffold; baseline (speedup 1.0000x reference)
import jax
import jax.numpy as jnp
from jax.experimental import pallas as pl


def kernel(x, edge_index, W, b):
    raise NotImplementedError("write your pallas kernel here")


# speedup vs baseline: 7.8435x; 7.8435x over previous
import functools

import jax
import jax.numpy as jnp
from jax import lax
from jax.experimental import pallas as pl
from jax.experimental.pallas import tpu as pltpu
from jax.experimental.pallas import tpu_sc as plsc

N = 10000
E = 320000
D = 128
DP = 144
NC = 2
NS = 16
NW = NC * NS
EPW = E // NW
K = 80
NCHUNK = EPW // K
RPS = N // NS

_mesh = plsc.VectorSubcoreMesh(core_axis_name="c", subcore_axis_name="s")


@functools.partial(
    pl.kernel,
    mesh=_mesh,
    out_type=jax.ShapeDtypeStruct((NC, N, DP), jnp.float32),
    scratch_types=[
        pltpu.VMEM((K,), jnp.int32),
        pltpu.VMEM((K,), jnp.int32),
        pltpu.VMEM((K, DP), jnp.float32),
        pltpu.VMEM_SHARED((N, DP), jnp.float32),
        pltpu.SemaphoreType.DMA,
    ],
    compiler_params=pltpu.CompilerParams(use_tc_tiling_on_sc=False),
)
def _sc_segment_sum(xe_hbm, src_hbm, dst_hbm, zeros_hbm, out_hbm,
                    src_v, dst_v, rows_v, acc, sem):
    c = lax.axis_index("c")
    s = lax.axis_index("s")
    wid = s * NC + c

    pltpu.sync_copy(zeros_hbm.at[pl.ds(s * RPS, RPS)],
                    acc.at[pl.ds(s * RPS, RPS)])
    plsc.subcore_barrier()

    base = wid * EPW

    def body(i, carry):
        off = pl.multiple_of(base + i * K, 8)
        pltpu.sync_copy(src_hbm.at[pl.ds(off, K)], src_v)
        pltpu.sync_copy(dst_hbm.at[pl.ds(off, K)], dst_v)
        pltpu.async_copy(xe_hbm.at[src_v], rows_v, sem).wait()
        pltpu.sync_copy(rows_v, acc.at[dst_v], add=True)
        return carry

    lax.fori_loop(0, NCHUNK, body, 0)
    plsc.subcore_barrier()

    pltpu.sync_copy(acc.at[pl.ds(s * RPS, RPS)],
                    out_hbm.at[c, pl.ds(s * RPS, RPS)])


BLK = 2000


def _tc_epilogue(x_ref, p_ref, wt_ref, b_ref, o_ref):
    p = p_ref[...]
    ssum = p[0] + p[1]
    cnt = ssum[:, D:D + 1]
    mean = ssum[:, :D] / jnp.maximum(cnt, 1.0)
    a = jnp.concatenate([x_ref[...], mean], axis=1)
    h = lax.dot_general(a, wt_ref[...], (((1,), (0,)), ((), ())),
                        preferred_element_type=jnp.float32)
    o_ref[...] = jnp.where(cnt > 0.0, h + b_ref[...], 0.0)


_epilogue_call = pl.pallas_call(
    _tc_epilogue,
    grid=(N // BLK,),
    in_specs=[
        pl.BlockSpec((BLK, D), lambda i: (i, 0)),
        pl.BlockSpec((NC, BLK, DP), lambda i: (0, i, 0)),
        pl.BlockSpec((2 * D, D), lambda i: (0, 0)),
        pl.BlockSpec((1, D), lambda i: (0, 0)),
    ],
    out_specs=pl.BlockSpec((BLK, D), lambda i: (i, 0)),
    out_shape=jax.ShapeDtypeStruct((N, D), jnp.float32),
)


def kernel(x, edge_index, W, b):
    src = edge_index[0]
    dst = edge_index[1]
    xe = jnp.concatenate(
        [x, jnp.ones((N, 1), jnp.float32), jnp.zeros((N, DP - D - 1), jnp.float32)],
        axis=1)
    zeros = jnp.zeros((N, DP), jnp.float32)
    partial = _sc_segment_sum(xe, src, dst, zeros)
    return _epilogue_call(x, partial, W.T, b.reshape(1, D))


# speedup vs baseline: 13.3092x; 1.6968x over previous
import functools

import jax
import jax.numpy as jnp
from jax import lax
from jax.experimental import pallas as pl
from jax.experimental.pallas import tpu as pltpu
from jax.experimental.pallas import tpu_sc as plsc

N = 10000
E = 320000
D = 128
DP = 144
NC = 2
NS = 16
NW = NC * NS
EPW = E // NW
K = 64
GC = 2
NCHUNK = 160
NG = NCHUNK // GC
EPT = NCHUNK * K
NPAD = EPT - EPW
NA = N + 16
RPZ = NA // NS
RPS = N // NS

_mesh = plsc.VectorSubcoreMesh(core_axis_name="c", subcore_axis_name="s")

_row_scratch = [pltpu.VMEM((K, DP), jnp.float32) for _ in range(2 * GC)]
_idx_scratch = [pltpu.VMEM((2, GC, K), jnp.int32) for _ in range(4)]
_sem_scratch = [pltpu.SemaphoreType.DMA for _ in range(2 * GC + 4)]


@functools.partial(
    pl.kernel,
    mesh=_mesh,
    out_type=jax.ShapeDtypeStruct((NC, N, DP), jnp.float32),
    scratch_types=[pltpu.VMEM_SHARED((NA, DP), jnp.float32)]
    + _row_scratch + _idx_scratch + _sem_scratch,
    compiler_params=pltpu.CompilerParams(use_tc_tiling_on_sc=False),
)
def _sc_segment_sum(xe_hbm, idx_hbm, zeros_hbm, out_hbm, acc, *scratch):
    rows = scratch[:2 * GC]
    idxb = scratch[2 * GC:2 * GC + 4]
    sems = scratch[2 * GC + 4:4 * GC + 4]
    isem = scratch[4 * GC + 4:]

    c = lax.axis_index("c")
    s = lax.axis_index("s")
    wid = s * NC + c

    def fire_idx(g, r):
        pltpu.async_copy(idx_hbm.at[wid, g], idxb[r], isem[r])

    def wait_idx(r):
        pltpu.make_async_copy(idx_hbm.at[0, 0], idxb[r], isem[r]).wait()

    def fire_b(g, p, r):
        for b in range(GC):
            pltpu.async_copy(xe_hbm.at[idxb[r].at[0, b]],
                             rows[GC * p + b], sems[GC * p + b])

    def fire_c(g, p, r):
        for b in range(GC):
            pltpu.async_copy(rows[GC * p + b], acc.at[idxb[r].at[1, b]],
                             sems[GC * p + b], add=True)

    def wait_rows(p):
        for b in range(GC):
            pltpu.make_async_copy(xe_hbm.at[pl.ds(0, K)],
                                  rows[GC * p + b], sems[GC * p + b]).wait()

    fire_idx(0, 0)
    fire_idx(1, 1)
    fire_idx(2, 2)
    wait_idx(0)
    fire_b(0, 0, 0)
    pltpu.sync_copy(zeros_hbm.at[pl.ds(s * RPZ, RPZ)],
                    acc.at[pl.ds(s * RPZ, RPZ)])
    plsc.subcore_barrier()

    wait_rows(0)
    fire_c(0, 0, 0)
    wait_idx(1)
    fire_b(1, 1, 1)
    fire_idx(3, 3)

    def group_body(g, p, r, rn, rf, last_idx, last_b):
        wait_rows(p)
        fire_c(g, p, r)
        if not last_b:
            wait_idx(rn)
        wait_rows(1 - p)
        if not last_b:
            fire_b(g + 1, 1 - p, rn)
        if not last_idx:
            fire_idx(g + 3, rf)

    def quad(i, carry):
        g = 1 + 4 * i
        group_body(g, 1, 1, 2, 0, False, False)
        group_body(g + 1, 0, 2, 3, 1, False, False)
        group_body(g + 2, 1, 3, 0, 2, False, False)
        group_body(g + 3, 0, 0, 1, 3, False, False)
        return carry

    lax.fori_loop(0, (NG - 4) // 4, quad, 0)

    group_body(NG - 3, 1, 1, 2, 0, True, False)
    group_body(NG - 2, 0, 2, 3, 1, True, False)
    group_body(NG - 1, 1, 3, 0, 2, True, True)
    wait_rows(1)
    plsc.subcore_barrier()

    pltpu.sync_copy(acc.at[pl.ds(s * RPS, RPS)],
                    out_hbm.at[c, pl.ds(s * RPS, RPS)])


BLK = 2000


def _tc_epilogue(x_ref, p_ref, wt_ref, b_ref, o_ref):
    p = p_ref[...]
    ssum = p[0] + p[1]
    cnt = ssum[:, D:D + 1]
    mean = ssum[:, :D] / jnp.maximum(cnt, 1.0)
    a = jnp.concatenate([x_ref[...], mean], axis=1)
    h = lax.dot_general(a, wt_ref[...], (((1,), (0,)), ((), ())),
                        preferred_element_type=jnp.float32)
    o_ref[...] = jnp.where(cnt > 0.0, h + b_ref[...], 0.0)


_epilogue_call = pl.pallas_call(
    _tc_epilogue,
    grid=(N // BLK,),
    in_specs=[
        pl.BlockSpec((BLK, D), lambda i: (i, 0)),
        pl.BlockSpec((NC, BLK, DP), lambda i: (0, i, 0)),
        pl.BlockSpec((2 * D, D), lambda i: (0, 0)),
        pl.BlockSpec((1, D), lambda i: (0, 0)),
    ],
    out_specs=pl.BlockSpec((BLK, D), lambda i: (i, 0)),
    out_shape=jax.ShapeDtypeStruct((N, D), jnp.float32),
)


def kernel(x, edge_index, W, b):
    src = edge_index[0]
    dst = edge_index[1]
    pad_src = jnp.broadcast_to((jnp.arange(NPAD, dtype=jnp.int32) * 41) % N,
                               (NW, NPAD))
    pad_dst = jnp.broadcast_to(N + (jnp.arange(NPAD, dtype=jnp.int32) % 16),
                               (NW, NPAD))
    src3 = jnp.concatenate([src.reshape(NW, EPW), pad_src], axis=1)
    dst3 = jnp.concatenate([dst.reshape(NW, EPW), pad_dst], axis=1)
    idx = jnp.stack([src3.reshape(NW, NG, GC, K),
                     dst3.reshape(NW, NG, GC, K)], axis=2)
    xe = jnp.concatenate(
        [x, jnp.ones((N, 1), jnp.float32), jnp.zeros((N, DP - D - 1), jnp.float32)],
        axis=1)
    zeros = jnp.zeros((NA, DP), jnp.float32)
    partial = _sc_segment_sum(xe, idx, zeros)
    return _epilogue_call(x, partial, W.T, b.reshape(1, D))


# speedup vs baseline: 15.9317x; 1.1970x over previous
import functools

import jax
import jax.numpy as jnp
from jax import lax
from jax.experimental import pallas as pl
from jax.experimental.pallas import tpu as pltpu
from jax.experimental.pallas import tpu_sc as plsc

N = 10000
E = 320000
D = 128
DP = 144
NC = 2
NS = 16
NW = NC * NS
EPW = E // NW
K = 80
GC = 1
NCHUNK = EPW // K
NG = NCHUNK // GC
RPS = N // NS

_mesh = plsc.VectorSubcoreMesh(core_axis_name="c", subcore_axis_name="s")

_row_scratch = [pltpu.VMEM((K, D), jnp.float32) for _ in range(2 * GC)]
_idx_scratch = [pltpu.VMEM((2, GC, K), jnp.int32) for _ in range(4)]
_sem_scratch = [pltpu.SemaphoreType.DMA for _ in range(2 * GC + 4)]


@functools.partial(
    pl.kernel,
    mesh=_mesh,
    out_type=(jax.ShapeDtypeStruct((NC, N, D), jnp.float32),
              jax.ShapeDtypeStruct((NC, N, 16), jnp.float32)),
    scratch_types=[pltpu.VMEM_SHARED((N, D), jnp.float32),
                   pltpu.VMEM_SHARED((N, 16), jnp.float32),
                   pltpu.VMEM((K, 16), jnp.float32)]
    + _row_scratch + _idx_scratch + _sem_scratch,
    compiler_params=pltpu.CompilerParams(use_tc_tiling_on_sc=False),
)
def _sc_segment_sum(x_hbm, edge_hbm, zeros_hbm, zeros16_hbm,
                    out_hbm, cnt_hbm, acc, cnt, ones_v, *scratch):
    rows = scratch[:2 * GC]
    idxb = scratch[2 * GC:2 * GC + 4]
    sems = scratch[2 * GC + 4:4 * GC + 4]
    isem = scratch[4 * GC + 4:]

    c = lax.axis_index("c")
    s = lax.axis_index("s")
    wid = s * NC + c
    base = wid * EPW

    def fire_idx(g, r):
        for b in range(GC):
            off = pl.multiple_of(base + (g * GC + b) * K, 8)
            pltpu.async_copy(edge_hbm.at[0, pl.ds(off, K)],
                             idxb[r].at[0, b], isem[r])
            pltpu.async_copy(edge_hbm.at[1, pl.ds(off, K)],
                             idxb[r].at[1, b], isem[r])

    def wait_idx(r):
        for _ in range(2 * GC):
            pltpu.make_async_copy(edge_hbm.at[0, pl.ds(0, K)],
                                  idxb[r].at[0, 0], isem[r]).wait()

    def fire_b(g, p, r):
        for b in range(GC):
            pltpu.async_copy(x_hbm.at[idxb[r].at[0, b]],
                             rows[GC * p + b], sems[GC * p + b])

    def fire_c(g, p, r):
        for b in range(GC):
            pltpu.async_copy(rows[GC * p + b], acc.at[idxb[r].at[1, b]],
                             sems[GC * p + b], add=True)
            pltpu.async_copy(ones_v, cnt.at[idxb[r].at[1, b]],
                             sems[GC * p + b], add=True)

    def wait_b(p):
        for b in range(GC):
            pltpu.make_async_copy(x_hbm.at[pl.ds(0, K)],
                                  rows[GC * p + b], sems[GC * p + b]).wait()

    def wait_c(p):
        for b in range(GC):
            pltpu.make_async_copy(rows[GC * p + b], acc.at[pl.ds(0, K)],
                                  sems[GC * p + b]).wait()
            pltpu.make_async_copy(ones_v, cnt.at[pl.ds(0, K)],
                                  sems[GC * p + b]).wait()

    one16 = jnp.full((16,), 1.0, jnp.float32)
    for rr in range(K):
        ones_v[rr, pl.ds(0, 16)] = one16

    fire_idx(0, 0)
    fire_idx(1, 1)
    fire_idx(2, 2)
    wait_idx(0)
    fire_b(0, 0, 0)
    pltpu.sync_copy(zeros_hbm.at[pl.ds(s * RPS, RPS)],
                    acc.at[pl.ds(s * RPS, RPS)])
    pltpu.sync_copy(zeros16_hbm.at[pl.ds(s * RPS, RPS)],
                    cnt.at[pl.ds(s * RPS, RPS)])
    plsc.subcore_barrier()

    wait_b(0)
    fire_c(0, 0, 0)
    wait_idx(1)
    fire_b(1, 1, 1)
    fire_idx(3, 3)

    def group_body(g, p, r, rn, rf, last_idx, last_b):
        wait_b(p)
        fire_c(g, p, r)
        if not last_b:
            wait_idx(rn)
        wait_c(1 - p)
        if not last_b:
            fire_b(g + 1, 1 - p, rn)
        if not last_idx:
            fire_idx(g + 3, rf)

    def quad(i, carry):
        g = 1 + 4 * i
        group_body(g, 1, 1, 2, 0, False, False)
        group_body(g + 1, 0, 2, 3, 1, False, False)
        group_body(g + 2, 1, 3, 0, 2, False, False)
        group_body(g + 3, 0, 0, 1, 3, False, False)
        return carry

    lax.fori_loop(0, (NG - 4) // 4, quad, 0)

    group_body(NG - 4, 1, 1, 2, 0, False, False)
    group_body(NG - 3, 0, 2, 3, 1, True, False)
    group_body(NG - 2, 1, 3, 0, 2, True, False)
    group_body(NG - 1, 0, 0, 1, 3, True, True)
    wait_c(0)
    plsc.subcore_barrier()

    pltpu.sync_copy(acc.at[pl.ds(s * RPS, RPS)],
                    out_hbm.at[c, pl.ds(s * RPS, RPS)])
    pltpu.sync_copy(cnt.at[pl.ds(s * RPS, RPS)],
                    cnt_hbm.at[c, pl.ds(s * RPS, RPS)])


BLK = 2000


def _tc_epilogue(x_ref, p_ref, q_ref, wt_ref, b_ref, o_ref):
    p = p_ref[...]
    q = q_ref[...]
    ssum = p[0] + p[1]
    cnt = (q[0] + q[1])[:, :1]
    mean = ssum / jnp.maximum(cnt, 1.0)
    a = jnp.concatenate([x_ref[...], mean], axis=1)
    h = lax.dot_general(a, wt_ref[...], (((1,), (0,)), ((), ())),
                        preferred_element_type=jnp.float32)
    o_ref[...] = jnp.where(cnt > 0.0, h + b_ref[...], 0.0)


_epilogue_call = pl.pallas_call(
    _tc_epilogue,
    grid=(N // BLK,),
    in_specs=[
        pl.BlockSpec((BLK, D), lambda i: (i, 0)),
        pl.BlockSpec((NC, BLK, D), lambda i: (0, i, 0)),
        pl.BlockSpec((NC, BLK, 16), lambda i: (0, i, 0)),
        pl.BlockSpec((2 * D, D), lambda i: (0, 0)),
        pl.BlockSpec((1, D), lambda i: (0, 0)),
    ],
    out_specs=pl.BlockSpec((BLK, D), lambda i: (i, 0)),
    out_shape=jax.ShapeDtypeStruct((N, D), jnp.float32),
)


def kernel(x, edge_index, W, b):
    zeros = jnp.zeros((N, D), jnp.float32)
    zeros16 = jnp.zeros((N, 16), jnp.float32)
    partial, counts = _sc_segment_sum(x, edge_index, zeros, zeros16)
    return _epilogue_call(x, partial, counts, W.T, b.reshape(1, D))


# speedup vs baseline: 15.9688x; 1.0023x over previous
import functools

import jax
import jax.numpy as jnp
from jax import lax
from jax.experimental import pallas as pl
from jax.experimental.pallas import tpu as pltpu
from jax.experimental.pallas import tpu_sc as plsc

N = 10000
E = 320000
D = 128
DP = 144
NC = 2
NS = 16
NW = NC * NS
EPW = E // NW
K = 80
GC = 1
NCHUNK = EPW // K
NG = NCHUNK // GC
RPS = N // NS

_mesh = plsc.VectorSubcoreMesh(core_axis_name="c", subcore_axis_name="s")

NR = 3
NI = 6
_row_scratch = [pltpu.VMEM((K, D), jnp.float32) for _ in range(NR)]
_idx_scratch = [pltpu.VMEM((2, K), jnp.int32) for _ in range(NI)]
_sem_scratch = [pltpu.SemaphoreType.DMA for _ in range(NR + NI)]


@functools.partial(
    pl.kernel,
    mesh=_mesh,
    out_type=(jax.ShapeDtypeStruct((NC, N, D), jnp.float32),
              jax.ShapeDtypeStruct((NC, N, 16), jnp.float32)),
    scratch_types=[pltpu.VMEM_SHARED((N, D), jnp.float32),
                   pltpu.VMEM_SHARED((N, 16), jnp.float32),
                   pltpu.VMEM((K, 16), jnp.float32)]
    + _row_scratch + _idx_scratch + _sem_scratch,
    compiler_params=pltpu.CompilerParams(use_tc_tiling_on_sc=False),
)
def _sc_segment_sum(x_hbm, edge_hbm, zeros_hbm, zeros16_hbm,
                    out_hbm, cnt_hbm, acc, cnt, ones_v, *scratch):
    rows = scratch[:NR]
    idxb = scratch[NR:NR + NI]
    sems = scratch[NR + NI:2 * NR + NI]
    isem = scratch[2 * NR + NI:]

    c = lax.axis_index("c")
    s = lax.axis_index("s")
    wid = s * NC + c
    base = wid * EPW

    def fire_idx(g, r):
        off = pl.multiple_of(base + g * K, 8)
        pltpu.async_copy(edge_hbm.at[0, pl.ds(off, K)], idxb[r].at[0], isem[r])
        pltpu.async_copy(edge_hbm.at[1, pl.ds(off, K)], idxb[r].at[1], isem[r])

    def wait_idx(r):
        for _ in range(2):
            pltpu.make_async_copy(edge_hbm.at[0, pl.ds(0, K)],
                                  idxb[r].at[0], isem[r]).wait()

    def fire_b(g, p, r):
        pltpu.async_copy(x_hbm.at[idxb[r].at[0]], rows[p], sems[p])

    def fire_c(g, p, r):
        pltpu.async_copy(rows[p], acc.at[idxb[r].at[1]], sems[p], add=True)
        pltpu.async_copy(ones_v, cnt.at[idxb[r].at[1]], sems[p], add=True)

    def wait_b(p):
        pltpu.make_async_copy(x_hbm.at[pl.ds(0, K)], rows[p], sems[p]).wait()

    def wait_c(p):
        pltpu.make_async_copy(rows[p], acc.at[pl.ds(0, K)], sems[p]).wait()
        pltpu.make_async_copy(ones_v, cnt.at[pl.ds(0, K)], sems[p]).wait()

    one16 = jnp.full((16,), 1.0, jnp.float32)
    for rr in range(K):
        ones_v[rr, pl.ds(0, 16)] = one16

    fire_idx(0, 0)
    fire_idx(1, 1)
    fire_idx(2, 2)
    wait_idx(0)
    fire_b(0, 0, 0)
    pltpu.sync_copy(zeros_hbm.at[pl.ds(s * RPS, RPS)],
                    acc.at[pl.ds(s * RPS, RPS)])
    pltpu.sync_copy(zeros16_hbm.at[pl.ds(s * RPS, RPS)],
                    cnt.at[pl.ds(s * RPS, RPS)])
    plsc.subcore_barrier()

    wait_b(0)
    fire_c(0, 0, 0)
    wait_idx(1)
    fire_b(1, 1, 1)
    fire_idx(3, 3)
    wait_b(1)
    fire_c(1, 1, 1)
    wait_idx(2)
    fire_b(2, 2, 2)
    fire_idx(4, 4)

    def group_body(g, a, ib, last_idx, last_b):
        wait_b(a)
        fire_c(g, a, ib)
        if not last_b:
            wait_idx((ib + 1) % NI)
        wait_c((a + 1) % NR)
        if not last_b:
            fire_b(g + 1, (a + 1) % NR, (ib + 1) % NI)
        if not last_idx:
            fire_idx(g + 3, (ib + 3) % NI)

    def sextet(i, carry):
        g = 2 + 6 * i
        for j in range(6):
            group_body(g + j, (2 + j) % NR, (2 + j) % NI, False, False)
        return carry

    lax.fori_loop(0, (NG - 5) // 6, sextet, 0)

    group_body(NG - 3, (NG - 3) % NR, (NG - 3) % NI, True, False)
    group_body(NG - 2, (NG - 2) % NR, (NG - 2) % NI, True, False)
    group_body(NG - 1, (NG - 1) % NR, (NG - 1) % NI, True, True)
    wait_c((NG - 2) % NR)
    wait_c((NG - 1) % NR)
    plsc.subcore_barrier()

    pltpu.sync_copy(acc.at[pl.ds(s * RPS, RPS)],
                    out_hbm.at[c, pl.ds(s * RPS, RPS)])
    pltpu.sync_copy(cnt.at[pl.ds(s * RPS, RPS)],
                    cnt_hbm.at[c, pl.ds(s * RPS, RPS)])


BLK = 2000


def _tc_epilogue(x_ref, p_ref, q_ref, wt_ref, b_ref, o_ref):
    p = p_ref[...]
    q = q_ref[...]
    ssum = p[0] + p[1]
    cnt = (q[0] + q[1])[:, :1]
    mean = ssum / jnp.maximum(cnt, 1.0)
    a = jnp.concatenate([x_ref[...], mean], axis=1)
    h = lax.dot_general(a, wt_ref[...], (((1,), (0,)), ((), ())),
                        preferred_element_type=jnp.float32)
    o_ref[...] = jnp.where(cnt > 0.0, h + b_ref[...], 0.0)


_epilogue_call = pl.pallas_call(
    _tc_epilogue,
    grid=(N // BLK,),
    in_specs=[
        pl.BlockSpec((BLK, D), lambda i: (i, 0)),
        pl.BlockSpec((NC, BLK, D), lambda i: (0, i, 0)),
        pl.BlockSpec((NC, BLK, 16), lambda i: (0, i, 0)),
        pl.BlockSpec((2 * D, D), lambda i: (0, 0)),
        pl.BlockSpec((1, D), lambda i: (0, 0)),
    ],
    out_specs=pl.BlockSpec((BLK, D), lambda i: (i, 0)),
    out_shape=jax.ShapeDtypeStruct((N, D), jnp.float32),
)


def kernel(x, edge_index, W, b):
    zeros = jnp.zeros((N, D), jnp.float32)
    zeros16 = jnp.zeros((N, 16), jnp.float32)
    partial, counts = _sc_segment_sum(x, edge_index, zeros, zeros16)
    return _epilogue_call(x, partial, counts, W.T, b.reshape(1, D))
